# baseline (speedup 1.0000x reference)
import functools

import jax
import jax.numpy as jnp
from jax import lax
from jax.experimental import pallas as pl
from jax.experimental.pallas import tpu as pltpu
from jax.experimental.pallas import tpu_sc as plsc

N_NODES = 10000
N_EDGES = 320000
N_TYPES = 4
N_GRAPHS = 128
H = 128
D_FEAT = 128
PE_DIM = 16

NC, NS = 2, 16
NW = NC * NS
EC = 128
E_ROWS = 2560
E_PAD = E_ROWS * EC
ROWS_PT = E_ROWS // NW
IB = 4
N_CHUNK = ROWS_PT // IB
N_AGG = 10208
STRIPE = N_AGG // NS

BN = 400
NB = N_NODES // BN

_HIGH = jax.lax.Precision.HIGHEST


def _rid_body(src_ref, attr_ref, rid_ref):
    rid_ref[...] = attr_ref[...] * N_NODES + src_ref[...]


def _compute_rid(src_p, attr_p):
    return pl.pallas_call(
        _rid_body,
        out_shape=jax.ShapeDtypeStruct((E_ROWS, EC), jnp.int32),
        grid=(E_ROWS // 256,),
        in_specs=[pl.BlockSpec((256, EC), lambda i: (i, 0)),
                  pl.BlockSpec((256, EC), lambda i: (i, 0))],
        out_specs=pl.BlockSpec((256, EC), lambda i: (i, 0)),
    )(src_p, attr_p)


def _table_body(x_ref, emb_ref, out_ref):
    out_ref[...] = jnp.maximum(
        x_ref[...][None, :, :] + emb_ref[...][:, None, :], 0.0)


def _build_table(x, emb, din):
    t = pl.pallas_call(
        _table_body,
        out_shape=jax.ShapeDtypeStruct((N_TYPES, N_NODES, din), jnp.float32),
        grid=(NB,),
        in_specs=[pl.BlockSpec((BN, din), lambda i: (i, 0)),
                  pl.BlockSpec((N_TYPES, din), lambda i: (0, 0))],
        out_specs=pl.BlockSpec((N_TYPES, BN, din), lambda i: (0, i, 0)),
    )(x, emb)
    return t.reshape(N_TYPES * N_NODES, din)


@functools.lru_cache(maxsize=None)
def _make_sc_agg(widths):
    n_tab = len(widths)
    mesh = plsc.VectorSubcoreMesh(core_axis_name="c", subcore_axis_name="s",
                                  num_cores=NC, num_subcores=NS)
    scratch = [pltpu.VMEM((2, IB, EC), jnp.int32),
               pltpu.VMEM((2, IB, EC), jnp.int32)]
    scratch += [pltpu.VMEM((2, EC, d), jnp.float32) for d in widths]
    scratch += [pltpu.VMEM_SHARED((N_AGG, d), jnp.float32) for d in widths]
    scratch += [pltpu.SemaphoreType.DMA] * n_tab
    out_type = tuple(jax.ShapeDtypeStruct((NC, N_AGG, d), jnp.float32)
                     for d in widths)

    def body(*refs):
        tabs = refs[:n_tab]
        rid_hbm = refs[n_tab]
        dst_hbm = refs[n_tab + 1]
        outs = refs[n_tab + 2: 2 * n_tab + 2]
        rid_v = refs[2 * n_tab + 2]
        dst_v = refs[2 * n_tab + 3]
        rows = refs[2 * n_tab + 4: 3 * n_tab + 4]
        aggs = refs[3 * n_tab + 4: 4 * n_tab + 4]
        sems = refs[4 * n_tab + 4: 5 * n_tab + 4]

        c = lax.axis_index("c")
        s = lax.axis_index("s")
        wid = s * NC + c
        base = wid * ROWS_PT

        def refill(chunk):
            slot = lax.rem(chunk, 2)
            pltpu.sync_copy(rid_hbm.at[pl.ds(base + chunk * IB, IB)],
                            rid_v.at[slot])
            pltpu.sync_copy(dst_hbm.at[pl.ds(base + chunk * IB, IB)],
                            dst_v.at[slot])

        def idx_row(ref, t):
            return ref.at[lax.rem(t // IB, 2), lax.rem(t, IB)]

        for i, d in enumerate(widths):
            def zrow(r, _, i=i, d=d):
                for j in range(d // 16):
                    rows[i][0, r, pl.ds(j * 16, 16)] = jnp.zeros(
                        (16,), jnp.float32)
                return 0
            lax.fori_loop(0, EC, zrow, 0)
            for k in range(STRIPE // EC):
                pltpu.sync_copy(rows[i].at[0],
                                aggs[i].at[pl.ds(s * STRIPE + k * EC, EC)])
            rem = STRIPE % EC
            if rem:
                pltpu.sync_copy(
                    rows[i].at[0, pl.ds(0, rem)],
                    aggs[i].at[pl.ds(s * STRIPE + (STRIPE // EC) * EC, rem)])
        plsc.subcore_barrier()

        refill(0)
        for i in range(n_tab):
            pltpu.async_copy(tabs[i].at[idx_row(rid_v, 0)], rows[i].at[0],
                             sems[i])

        def step(g, _):
            b = lax.rem(g, 2)

            @pl.when(jnp.logical_and(lax.rem(g, IB) == 0,
                                     g // IB + 1 < N_CHUNK))
            def _():
                refill(g // IB + 1)

            for i in range(n_tab):
                pltpu.make_async_copy(tabs[i].at[idx_row(rid_v, g)],
                                      rows[i].at[b], sems[i]).wait()

            @pl.when(g + 1 < ROWS_PT)
            def _():
                for i in range(n_tab):
                    pltpu.async_copy(tabs[i].at[idx_row(rid_v, g + 1)],
                                     rows[i].at[1 - b], sems[i])

            for i in range(n_tab):
                pltpu.sync_copy(rows[i].at[b], aggs[i].at[idx_row(dst_v, g)],
                                add=True)
            return 0

        lax.fori_loop(0, ROWS_PT, step, 0)
        plsc.subcore_barrier()

        for i in range(n_tab):
            pltpu.sync_copy(aggs[i].at[pl.ds(s * STRIPE, STRIPE)],
                            outs[i].at[c, pl.ds(s * STRIPE, STRIPE)])

    return pl.kernel(body, out_type=out_type, mesh=mesh,
                     scratch_types=scratch,
                     compiler_params=pltpu.CompilerParams(
                         use_tc_tiling_on_sc=False))


def _mlp_body(residual, n_agg, x_ref, *refs):
    agg_refs = refs[:n_agg]
    eps_ref, w1_ref, b1_ref, w2_ref, b2_ref, out_ref = refs[n_agg:]
    e = eps_ref[0, 0]
    parts = [a[0] + a[1] for a in agg_refs]
    agg = parts[0] if n_agg == 1 else jnp.concatenate(parts, axis=1)
    z = (1.0 + e) * x_ref[...] + agg
    h = jnp.dot(z.astype(jnp.bfloat16), w1_ref[...].astype(jnp.bfloat16),
                preferred_element_type=jnp.float32) + b1_ref[...]
    h = jnp.maximum(h, 0.0)
    h = jnp.dot(h.astype(jnp.bfloat16), w2_ref[...].astype(jnp.bfloat16),
                preferred_element_type=jnp.float32) + b2_ref[...]
    if residual:
        h = h + x_ref[...]
    out_ref[...] = h


def _layer_mlp(x, aggs, widths, eps2d, w1, b1, w2, b2, residual):
    din = sum(widths)
    agg_specs = [pl.BlockSpec((NC, BN, d), lambda i: (0, i, 0))
                 for d in widths]
    return pl.pallas_call(
        functools.partial(_mlp_body, residual, len(widths)),
        out_shape=jax.ShapeDtypeStruct((N_NODES, H), jnp.float32),
        grid=(NB,),
        in_specs=[pl.BlockSpec((BN, din), lambda i: (i, 0))] + agg_specs + [
            pl.BlockSpec((1, 1), lambda i: (0, 0)),
            pl.BlockSpec((din, H), lambda i: (0, 0)),
            pl.BlockSpec((1, H), lambda i: (0, 0)),
            pl.BlockSpec((H, H), lambda i: (0, 0)),
            pl.BlockSpec((1, H), lambda i: (0, 0)),
        ],
        out_specs=pl.BlockSpec((BN, H), lambda i: (i, 0)),
    )(x, *aggs, eps2d, w1, b1.reshape(1, H), w2, b2.reshape(1, H))


def _head_body(x_ref, sn_ref, bt_ref, w1_ref, b1_ref, w2r_ref, b2_ref,
               out_ref, sums_ref, cnts_ref):
    i = pl.program_id(0)

    @pl.when(i == 0)
    def _():
        sums_ref[...] = jnp.zeros_like(sums_ref)
        cnts_ref[...] = jnp.zeros_like(cnts_ref)

    y = x_ref[...] * sn_ref[...]
    gids = jax.lax.broadcasted_iota(jnp.int32, (1, N_GRAPHS), 1)
    oh = (bt_ref[...] == gids).astype(jnp.float32)
    sums_ref[...] += jax.lax.dot_general(
        oh, y, (((0,), (0,)), ((), ())), precision=_HIGH,
        preferred_element_type=jnp.float32)
    cnts_ref[...] += jax.lax.dot_general(
        oh, jnp.ones((BN, 1), jnp.float32), (((0,), (0,)), ((), ())),
        precision=_HIGH, preferred_element_type=jnp.float32)

    @pl.when(i == NB - 1)
    def _():
        pooled = sums_ref[...] / jnp.maximum(cnts_ref[...], 1.0)
        h = jnp.dot(pooled.astype(jnp.bfloat16),
                    w1_ref[...].astype(jnp.bfloat16),
                    preferred_element_type=jnp.float32) + b1_ref[...]
        h = jnp.maximum(h, 0.0)
        hb = h.astype(jnp.bfloat16).astype(jnp.float32)
        wb = w2r_ref[...].astype(jnp.bfloat16).astype(jnp.float32)
        out_ref[...] = (jnp.sum(hb * wb, axis=1, keepdims=True)
                        + b2_ref[0, 0])


def _pool_head(x, snorm2d, batch2d, w1, b1, w2row, b2s):
    return pl.pallas_call(
        _head_body,
        out_shape=jax.ShapeDtypeStruct((N_GRAPHS, 1), jnp.float32),
        grid=(NB,),
        in_specs=[
            pl.BlockSpec((BN, H), lambda i: (i, 0)),
            pl.BlockSpec((BN, 1), lambda i: (i, 0)),
            pl.BlockSpec((BN, 1), lambda i: (i, 0)),
            pl.BlockSpec((H, H), lambda i: (0, 0)),
            pl.BlockSpec((1, H), lambda i: (0, 0)),
            pl.BlockSpec((1, H), lambda i: (0, 0)),
            pl.BlockSpec((1, 1), lambda i: (0, 0)),
        ],
        out_specs=pl.BlockSpec((N_GRAPHS, 1), lambda i: (0, 0)),
        scratch_shapes=[pltpu.VMEM((N_GRAPHS, H), jnp.float32),
                        pltpu.VMEM((N_GRAPHS, 1), jnp.float32)],
    )(x, snorm2d, batch2d, w1, b1, w2row, b2s)


def _jnp_agg(widths):
    def run(*args):
        tabs = args[:len(widths)]
        rid, dstp = args[len(widths)], args[len(widths) + 1]
        rid_f = rid.reshape(-1)
        dst_f = dstp.reshape(-1)
        outs = []
        for tab in tabs:
            rows = tab[rid_f]
            agg = jax.ops.segment_sum(rows, dst_f, num_segments=N_AGG)
            outs.append(jnp.stack([agg, jnp.zeros_like(agg)]))
        return tuple(outs)
    return run


def kernel(X_n, edge_index, edge_attr, PE, snorm, batch, params):
    src = edge_index[0].astype(jnp.int32)
    dst = edge_index[1].astype(jnp.int32)
    attr = edge_attr.astype(jnp.int32)

    pad = E_PAD - N_EDGES
    pi = jnp.arange(pad, dtype=jnp.int32)
    src_p = jnp.concatenate([src, pi % N_NODES]).reshape(E_ROWS, EC)
    attr_p = jnp.concatenate([attr, pi % N_TYPES]).reshape(E_ROWS, EC)
    dst_p = jnp.concatenate(
        [dst, N_NODES + pi % (N_AGG - N_NODES)]).reshape(E_ROWS, EC)
    rid = _compute_rid(src_p, attr_p)

    layers = params["layers"]

    lp = layers[0]
    emb = lp["edge_emb"]
    ta = _build_table(X_n, emb[:, :D_FEAT], D_FEAT)
    tb = _build_table(PE, emb[:, D_FEAT:], PE_DIM)
    agg_a, agg_b = _jnp_agg((D_FEAT, PE_DIM))(ta, tb, rid, dst_p)
    x0 = jnp.concatenate([X_n, PE], axis=-1)
    eps2d = jnp.reshape(lp["eps"], (1, 1))
    x = _layer_mlp(x0, [agg_a, agg_b], (D_FEAT, PE_DIM), eps2d,
                   lp["W1"], lp["b1"], lp["W2"], lp["b2"], residual=False)

    for lp in layers[1:]:
        t = _build_table(x, lp["edge_emb"], H)
        (agg,) = _jnp_agg((H,))(t, rid, dst_p)
        eps2d = jnp.reshape(lp["eps"], (1, 1))
        x = _layer_mlp(x, [agg], (H,), eps2d,
                       lp["W1"], lp["b1"], lp["W2"], lp["b2"], residual=True)

    hp = params["head"]
    y = _pool_head(x, snorm.reshape(N_NODES, 1),
                   batch.astype(jnp.int32).reshape(N_NODES, 1),
                   hp["W1"], hp["b1"].reshape(1, H),
                   hp["W2"].reshape(1, H), hp["b2"].reshape(1, 1))
    return y.reshape(N_GRAPHS)


# speedup vs baseline: 10.7776x; 10.7776x over previous
import functools

import jax
import jax.numpy as jnp
from jax import lax
from jax.experimental import pallas as pl
from jax.experimental.pallas import tpu as pltpu
from jax.experimental.pallas import tpu_sc as plsc

N_NODES = 10000
N_EDGES = 320000
N_TYPES = 4
N_GRAPHS = 128
H = 128
D_FEAT = 128
PE_DIM = 16

NC, NS = 2, 16
NW = NC * NS
EC = 128
E_ROWS = 2560
E_PAD = E_ROWS * EC
ROWS_PT = E_ROWS // NW
IB = 4
N_CHUNK = ROWS_PT // IB
N_AGG = 10208
STRIPE = N_AGG // NS

BN = 400
NB = N_NODES // BN

_HIGH = jax.lax.Precision.HIGHEST


def _rid_body(src_ref, attr_ref, rid_ref):
    rid_ref[...] = attr_ref[...] * N_NODES + src_ref[...]


def _compute_rid(src_p, attr_p):
    return pl.pallas_call(
        _rid_body,
        out_shape=jax.ShapeDtypeStruct((E_ROWS, EC), jnp.int32),
        grid=(E_ROWS // 256,),
        in_specs=[pl.BlockSpec((256, EC), lambda i: (i, 0)),
                  pl.BlockSpec((256, EC), lambda i: (i, 0))],
        out_specs=pl.BlockSpec((256, EC), lambda i: (i, 0)),
    )(src_p, attr_p)


def _table_body(x_ref, emb_ref, out_ref):
    out_ref[...] = jnp.maximum(
        x_ref[...][None, :, :] + emb_ref[...][:, None, :], 0.0)


def _build_table(x, emb, din):
    t = pl.pallas_call(
        _table_body,
        out_shape=jax.ShapeDtypeStruct((N_TYPES, N_NODES, din), jnp.float32),
        grid=(NB,),
        in_specs=[pl.BlockSpec((BN, din), lambda i: (i, 0)),
                  pl.BlockSpec((N_TYPES, din), lambda i: (0, 0))],
        out_specs=pl.BlockSpec((N_TYPES, BN, din), lambda i: (0, i, 0)),
    )(x, emb)
    return t.reshape(N_TYPES * N_NODES, din)


@functools.lru_cache(maxsize=None)
def _make_sc_agg(widths):
    n_tab = len(widths)
    mesh = plsc.VectorSubcoreMesh(core_axis_name="c", subcore_axis_name="s",
                                  num_cores=NC, num_subcores=NS)
    scratch = [pltpu.VMEM((2, IB, EC), jnp.int32),
               pltpu.VMEM((2, IB, EC), jnp.int32)]
    scratch += [pltpu.VMEM((2, EC, d), jnp.float32) for d in widths]
    scratch += [pltpu.VMEM_SHARED((N_AGG, d), jnp.float32) for d in widths]
    scratch += [pltpu.SemaphoreType.DMA] * n_tab
    out_type = tuple(jax.ShapeDtypeStruct((NC, N_AGG, d), jnp.float32)
                     for d in widths)

    def body(*refs):
        tabs = refs[:n_tab]
        rid_hbm = refs[n_tab]
        dst_hbm = refs[n_tab + 1]
        outs = refs[n_tab + 2: 2 * n_tab + 2]
        rid_v = refs[2 * n_tab + 2]
        dst_v = refs[2 * n_tab + 3]
        rows = refs[2 * n_tab + 4: 3 * n_tab + 4]
        aggs = refs[3 * n_tab + 4: 4 * n_tab + 4]
        sems = refs[4 * n_tab + 4: 5 * n_tab + 4]

        c = lax.axis_index("c")
        s = lax.axis_index("s")
        wid = s * NC + c
        base = wid * ROWS_PT

        def refill(chunk):
            slot = lax.rem(chunk, 2)
            pltpu.sync_copy(rid_hbm.at[pl.ds(base + chunk * IB, IB)],
                            rid_v.at[slot])
            pltpu.sync_copy(dst_hbm.at[pl.ds(base + chunk * IB, IB)],
                            dst_v.at[slot])

        def idx_row(ref, t):
            return ref.at[lax.rem(t // IB, 2), lax.rem(t, IB)]

        for i, d in enumerate(widths):
            def zrow(r, _, i=i, d=d):
                for j in range(d // 16):
                    rows[i][0, r, pl.ds(j * 16, 16)] = jnp.zeros(
                        (16,), jnp.float32)
                return 0
            lax.fori_loop(0, EC, zrow, 0)
            for k in range(STRIPE // EC):
                pltpu.sync_copy(rows[i].at[0],
                                aggs[i].at[pl.ds(s * STRIPE + k * EC, EC)])
            rem = STRIPE % EC
            if rem:
                pltpu.sync_copy(
                    rows[i].at[0, pl.ds(0, rem)],
                    aggs[i].at[pl.ds(s * STRIPE + (STRIPE // EC) * EC, rem)])
        plsc.subcore_barrier()

        refill(0)
        for i in range(n_tab):
            pltpu.async_copy(tabs[i].at[idx_row(rid_v, 0)], rows[i].at[0],
                             sems[i])

        def step(g, _):
            b = lax.rem(g, 2)

            @pl.when(jnp.logical_and(lax.rem(g, IB) == 0,
                                     g // IB + 1 < N_CHUNK))
            def _():
                refill(g // IB + 1)

            for i in range(n_tab):
                pltpu.make_async_copy(tabs[i].at[idx_row(rid_v, g)],
                                      rows[i].at[b], sems[i]).wait()

            @pl.when(g + 1 < ROWS_PT)
            def _():
                for i in range(n_tab):
                    pltpu.async_copy(tabs[i].at[idx_row(rid_v, g + 1)],
                                     rows[i].at[1 - b], sems[i])

            for i in range(n_tab):
                pltpu.sync_copy(rows[i].at[b], aggs[i].at[idx_row(dst_v, g)],
                                add=True)
            return 0

        lax.fori_loop(0, ROWS_PT, step, 0)
        plsc.subcore_barrier()

        for i in range(n_tab):
            pltpu.sync_copy(aggs[i].at[pl.ds(s * STRIPE, STRIPE)],
                            outs[i].at[c, pl.ds(s * STRIPE, STRIPE)])

    return pl.kernel(body, out_type=out_type, mesh=mesh,
                     scratch_types=scratch,
                     compiler_params=pltpu.CompilerParams(
                         use_tc_tiling_on_sc=False))


def _mlp_body(residual, n_agg, x_ref, *refs):
    agg_refs = refs[:n_agg]
    eps_ref, w1_ref, b1_ref, w2_ref, b2_ref, out_ref = refs[n_agg:]
    e = eps_ref[0, 0]
    parts = [a[0] + a[1] for a in agg_refs]
    agg = parts[0] if n_agg == 1 else jnp.concatenate(parts, axis=1)
    z = (1.0 + e) * x_ref[...] + agg
    h = jnp.dot(z.astype(jnp.bfloat16), w1_ref[...].astype(jnp.bfloat16),
                preferred_element_type=jnp.float32) + b1_ref[...]
    h = jnp.maximum(h, 0.0)
    h = jnp.dot(h.astype(jnp.bfloat16), w2_ref[...].astype(jnp.bfloat16),
                preferred_element_type=jnp.float32) + b2_ref[...]
    if residual:
        h = h + x_ref[...]
    out_ref[...] = h


def _layer_mlp(x, aggs, widths, eps2d, w1, b1, w2, b2, residual):
    din = sum(widths)
    agg_specs = [pl.BlockSpec((NC, BN, d), lambda i: (0, i, 0))
                 for d in widths]
    return pl.pallas_call(
        functools.partial(_mlp_body, residual, len(widths)),
        out_shape=jax.ShapeDtypeStruct((N_NODES, H), jnp.float32),
        grid=(NB,),
        in_specs=[pl.BlockSpec((BN, din), lambda i: (i, 0))] + agg_specs + [
            pl.BlockSpec((1, 1), lambda i: (0, 0)),
            pl.BlockSpec((din, H), lambda i: (0, 0)),
            pl.BlockSpec((1, H), lambda i: (0, 0)),
            pl.BlockSpec((H, H), lambda i: (0, 0)),
            pl.BlockSpec((1, H), lambda i: (0, 0)),
        ],
        out_specs=pl.BlockSpec((BN, H), lambda i: (i, 0)),
    )(x, *aggs, eps2d, w1, b1.reshape(1, H), w2, b2.reshape(1, H))


def _head_body(x_ref, sn_ref, bt_ref, w1_ref, b1_ref, w2r_ref, b2_ref,
               out_ref, sums_ref, cnts_ref):
    i = pl.program_id(0)

    @pl.when(i == 0)
    def _():
        sums_ref[...] = jnp.zeros_like(sums_ref)
        cnts_ref[...] = jnp.zeros_like(cnts_ref)

    y = x_ref[...] * sn_ref[...]
    gids = jax.lax.broadcasted_iota(jnp.int32, (1, N_GRAPHS), 1)
    oh = (bt_ref[...] == gids).astype(jnp.float32)
    sums_ref[...] += jax.lax.dot_general(
        oh, y, (((0,), (0,)), ((), ())), precision=_HIGH,
        preferred_element_type=jnp.float32)
    cnts_ref[...] += jax.lax.dot_general(
        oh, jnp.ones((BN, 1), jnp.float32), (((0,), (0,)), ((), ())),
        precision=_HIGH, preferred_element_type=jnp.float32)

    @pl.when(i == NB - 1)
    def _():
        pooled = sums_ref[...] / jnp.maximum(cnts_ref[...], 1.0)
        h = jnp.dot(pooled.astype(jnp.bfloat16),
                    w1_ref[...].astype(jnp.bfloat16),
                    preferred_element_type=jnp.float32) + b1_ref[...]
        h = jnp.maximum(h, 0.0)
        hb = h.astype(jnp.bfloat16).astype(jnp.float32)
        wb = w2r_ref[...].astype(jnp.bfloat16).astype(jnp.float32)
        out_ref[...] = (jnp.sum(hb * wb, axis=1, keepdims=True)
                        + b2_ref[0, 0])


def _pool_head(x, snorm2d, batch2d, w1, b1, w2row, b2s):
    return pl.pallas_call(
        _head_body,
        out_shape=jax.ShapeDtypeStruct((N_GRAPHS, 1), jnp.float32),
        grid=(NB,),
        in_specs=[
            pl.BlockSpec((BN, H), lambda i: (i, 0)),
            pl.BlockSpec((BN, 1), lambda i: (i, 0)),
            pl.BlockSpec((BN, 1), lambda i: (i, 0)),
            pl.BlockSpec((H, H), lambda i: (0, 0)),
            pl.BlockSpec((1, H), lambda i: (0, 0)),
            pl.BlockSpec((1, H), lambda i: (0, 0)),
            pl.BlockSpec((1, 1), lambda i: (0, 0)),
        ],
        out_specs=pl.BlockSpec((N_GRAPHS, 1), lambda i: (0, 0)),
        scratch_shapes=[pltpu.VMEM((N_GRAPHS, H), jnp.float32),
                        pltpu.VMEM((N_GRAPHS, 1), jnp.float32)],
    )(x, snorm2d, batch2d, w1, b1, w2row, b2s)


def kernel(X_n, edge_index, edge_attr, PE, snorm, batch, params):
    src = edge_index[0].astype(jnp.int32)
    dst = edge_index[1].astype(jnp.int32)
    attr = edge_attr.astype(jnp.int32)

    pad = E_PAD - N_EDGES
    pi = jnp.arange(pad, dtype=jnp.int32)
    src_p = jnp.concatenate([src, pi % N_NODES]).reshape(E_ROWS, EC)
    attr_p = jnp.concatenate([attr, pi % N_TYPES]).reshape(E_ROWS, EC)
    dst_p = jnp.concatenate(
        [dst, N_NODES + pi % (N_AGG - N_NODES)]).reshape(E_ROWS, EC)
    rid = _compute_rid(src_p, attr_p)

    layers = params["layers"]

    lp = layers[0]
    emb = lp["edge_emb"]
    ta = _build_table(X_n, emb[:, :D_FEAT], D_FEAT)
    tb = _build_table(PE, emb[:, D_FEAT:], PE_DIM)
    agg_a, agg_b = _make_sc_agg((D_FEAT, PE_DIM))(ta, tb, rid, dst_p)
    x0 = jnp.concatenate([X_n, PE], axis=-1)
    eps2d = jnp.reshape(lp["eps"], (1, 1))
    x = _layer_mlp(x0, [agg_a, agg_b], (D_FEAT, PE_DIM), eps2d,
                   lp["W1"], lp["b1"], lp["W2"], lp["b2"], residual=False)

    for lp in layers[1:]:
        t = _build_table(x, lp["edge_emb"], H)
        (agg,) = _make_sc_agg((H,))(t, rid, dst_p)
        eps2d = jnp.reshape(lp["eps"], (1, 1))
        x = _layer_mlp(x, [agg], (H,), eps2d,
                       lp["W1"], lp["b1"], lp["W2"], lp["b2"], residual=True)

    hp = params["head"]
    y = _pool_head(x, snorm.reshape(N_NODES, 1),
                   batch.astype(jnp.int32).reshape(N_NODES, 1),
                   hp["W1"], hp["b1"].reshape(1, H),
                   hp["W2"].reshape(1, H), hp["b2"].reshape(1, 1))
    return y.reshape(N_GRAPHS)
